# baseline (speedup 1.0000x reference)
import jax
import jax.numpy as jnp
from jax import lax
from jax.experimental import pallas as pl
from jax.experimental.pallas import tpu as pltpu
from jax.experimental.pallas import tpu_sc as plsc

N = 10000
E = 320000
F = 128
H = 8
C = 16
HC = H * C
R = 1000
NC = 2
NS = 16
NW = NC * NS
EPT = E // NW
B = 80
NCHUNK = EPT // B
NPAD = 10112
RPT = NPAD // NS


def _tc1_body(x_ref, w_ref, asrc_ref, adst_ref, h_ref, asrc2_ref, adst2_ref):
    h = jnp.dot(x_ref[...], w_ref[...], preferred_element_type=jnp.float32)
    rows = lax.broadcasted_iota(jnp.int32, (HC, H), 0) // C
    cols = lax.broadcasted_iota(jnp.int32, (HC, H), 1)
    b8 = (rows == cols).astype(jnp.float32)
    a_s = jnp.dot(h * asrc_ref[...], b8, preferred_element_type=jnp.float32)
    a_d = jnp.dot(h * adst_ref[...], b8, preferred_element_type=jnp.float32)
    h_ref[...] = h
    asrc2_ref[...] = jnp.concatenate([a_s, a_s], axis=1)
    adst2_ref[...] = jnp.concatenate([a_d, a_d], axis=1)


def _stage1(x, w_lin, att_src_flat, att_dst_flat):
    return pl.pallas_call(
        _tc1_body,
        grid=(N // R,),
        in_specs=[
            pl.BlockSpec((R, F), lambda i: (i, 0)),
            pl.BlockSpec((F, HC), lambda i: (0, 0)),
            pl.BlockSpec((1, HC), lambda i: (0, 0)),
            pl.BlockSpec((1, HC), lambda i: (0, 0)),
        ],
        out_specs=[
            pl.BlockSpec((R, HC), lambda i: (i, 0)),
            pl.BlockSpec((R, C), lambda i: (i, 0)),
            pl.BlockSpec((R, C), lambda i: (i, 0)),
        ],
        out_shape=[
            jax.ShapeDtypeStruct((N, HC), jnp.float32),
            jax.ShapeDtypeStruct((N, C), jnp.float32),
            jax.ShapeDtypeStruct((N, C), jnp.float32),
        ],
    )(x, w_lin, att_src_flat, att_dst_flat)


def _bcast_lane(v, j):
    idx = jnp.full((C,), j, jnp.int32)
    return v.at[idx].get(mode="promise_in_bounds")


def _sc_body(h_hbm, as2_hbm, ad2_hbm, ei_hbm, zm_hbm, zw_hbm, outm_hbm, outw_hbm,
             idxr, hring, asr, adr, wring, acc_m, acc_w,
             sg0, sg1, ss0, ss1,
             si0, si1, si2, si3, si4, si5, si6, si7):
    sem_g = [sg0, sg1]
    sem_s = [ss0, ss1]
    sem_i = [si0, si1, si2, si3, si4, si5, si6, si7]
    cid = lax.axis_index("c")
    sid = lax.axis_index("s")
    wid = sid * NC + cid

    pltpu.sync_copy(zm_hbm, acc_m.at[pl.ds(sid * RPT, RPT)])
    pltpu.sync_copy(zw_hbm, acc_w.at[pl.ds(sid * RPT, RPT)])
    plsc.subcore_barrier()

    def issue_i(c, j):
        pltpu.async_copy(ei_hbm.at[0, wid, c], idxr.at[j, 0], sem_i[j])
        pltpu.async_copy(ei_hbm.at[1, wid, c], idxr.at[j, 1], sem_i[j])

    def wait_i(c, j):
        pltpu.make_async_copy(ei_hbm.at[0, wid, c], idxr.at[j, 0],
                              sem_i[j]).wait()
        pltpu.make_async_copy(ei_hbm.at[1, wid, c], idxr.at[j, 1],
                              sem_i[j]).wait()

    def issue_g(j, b):
        pltpu.async_copy(h_hbm.at[idxr.at[j, 0]], hring.at[b], sem_g[b])
        pltpu.async_copy(as2_hbm.at[idxr.at[j, 0]], asr.at[b], sem_g[b])
        pltpu.async_copy(ad2_hbm.at[idxr.at[j, 1]], adr.at[b], sem_g[b])

    def wait_g(j, b):
        pltpu.make_async_copy(h_hbm.at[idxr.at[j, 0]], hring.at[b],
                              sem_g[b]).wait()
        pltpu.make_async_copy(as2_hbm.at[idxr.at[j, 0]], asr.at[b],
                              sem_g[b]).wait()
        pltpu.make_async_copy(ad2_hbm.at[idxr.at[j, 1]], adr.at[b],
                              sem_g[b]).wait()

    def issue_s(j, b):
        pltpu.async_copy(hring.at[b], acc_m.at[idxr.at[j, 1]], sem_s[b],
                         add=True)
        pltpu.async_copy(wring.at[b], acc_w.at[idxr.at[j, 1]], sem_s[b],
                         add=True)

    def wait_s(j, b):
        pltpu.make_async_copy(hring.at[b], acc_m.at[idxr.at[j, 1]],
                              sem_s[b]).wait()
        pltpu.make_async_copy(wring.at[b], acc_w.at[idxr.at[j, 1]],
                              sem_s[b]).wait()

    def compute(b):
        @plsc.parallel_loop(0, B, unroll=4)
        def edge_body(e):
            logit = asr[b, e, :] + adr[b, e, :]
            logit = jnp.where(logit > 0, logit, 0.2 * logit)
            w = jnp.exp(logit)
            wring[b, e, :] = w
            for hd in range(H):
                bc = _bcast_lane(w, hd)
                hring[b, e, pl.ds(hd * C, C)] = hring[b, e, pl.ds(hd * C, C)] * bc

    def step(c, m, first=False, do_g=True, do_i=True):
        b = m % 2
        wait_g(m % 8, b)
        if not first:
            wait_s((m - 1) % 8, (m - 1) % 2)
        if do_g:
            wait_i(c + 1, (m + 1) % 8)
            issue_g((m + 1) % 8, (m + 1) % 2)
        compute(b)
        issue_s(m % 8, b)
        if do_i:
            issue_i(c + 4, (m + 4) % 8)

    for c in range(4):
        issue_i(c, c)
    wait_i(0, 0)
    issue_g(0, 0)
    step(0, 0, first=True)
    for c in range(1, 8):
        step(c, c)

    def octo(i, carry):
        for o in range(8):
            step(8 * i + o, o)
        return carry

    lax.fori_loop(1, (NCHUNK - 5) // 8, octo, 0)

    for c in range(NCHUNK - 5, NCHUNK):
        step(c, c % 8, do_g=(c + 1 <= NCHUNK - 1), do_i=(c + 4 <= NCHUNK - 1))
    wait_s((NCHUNK - 1) % 8, (NCHUNK - 1) % 2)
    plsc.subcore_barrier()

    for k in range(RPT // B):
        r0 = sid * RPT + k * B
        pltpu.sync_copy(acc_m.at[pl.ds(r0, B)], hring.at[k % 2])
        pltpu.sync_copy(hring.at[k % 2], outm_hbm.at[cid, pl.ds(r0, B)])
        pltpu.sync_copy(acc_w.at[pl.ds(r0, B)], wring.at[k % 2])
        pltpu.sync_copy(wring.at[k % 2], outw_hbm.at[cid, pl.ds(r0, B)])
    rt = RPT - (RPT // B) * B
    if rt:
        r0 = sid * RPT + (RPT // B) * B
        pltpu.sync_copy(acc_m.at[pl.ds(r0, rt)], hring.at[0, pl.ds(0, rt)])
        pltpu.sync_copy(hring.at[0, pl.ds(0, rt)],
                        outm_hbm.at[cid, pl.ds(r0, rt)])
        pltpu.sync_copy(acc_w.at[pl.ds(r0, rt)], wring.at[0, pl.ds(0, rt)])
        pltpu.sync_copy(wring.at[0, pl.ds(0, rt)],
                        outw_hbm.at[cid, pl.ds(r0, rt)])


def _stage2(h, as2, ad2, ei, zm, zw):
    mesh = plsc.VectorSubcoreMesh(core_axis_name="c", subcore_axis_name="s")
    k = pl.kernel(
        _sc_body,
        out_type=[
            jax.ShapeDtypeStruct((NC, NPAD, HC), jnp.float32),
            jax.ShapeDtypeStruct((NC, NPAD, C), jnp.float32),
        ],
        mesh=mesh,
        compiler_params=pltpu.CompilerParams(use_tc_tiling_on_sc=False),
        scratch_types=[
            pltpu.VMEM((8, 2, B), jnp.int32),
            pltpu.VMEM((2, B, HC), jnp.float32),
            pltpu.VMEM((2, B, C), jnp.float32),
            pltpu.VMEM((2, B, C), jnp.float32),
            pltpu.VMEM((2, B, C), jnp.float32),
            pltpu.VMEM_SHARED((NPAD, HC), jnp.float32),
            pltpu.VMEM_SHARED((NPAD, C), jnp.float32),
        ] + [pltpu.SemaphoreType.DMA] * 12,
    )
    return k(h, as2, ad2, ei, zm, zw)


def _tc3_body(pm_ref, pw_ref, h_ref, as2_ref, ad2_ref, x_ref, wff_ref, bff_ref,
              bconv_ref, g_ref, be_ref, o_ref):
    msg = pm_ref[0] + pm_ref[1]
    h = h_ref[...]
    l8 = as2_ref[:, :H] + ad2_ref[:, :H]
    w8 = jnp.exp(jnp.where(l8 > 0, l8, 0.2 * l8))
    s8 = pw_ref[0, :, :H] + pw_ref[1, :, :H] + w8
    rows_i = lax.broadcasted_iota(jnp.int32, (H, HC), 0)
    cols_i = lax.broadcasted_iota(jnp.int32, (H, HC), 1) // C
    bw = (rows_i == cols_i).astype(jnp.float32)
    w128 = jnp.dot(w8, bw, preferred_element_type=jnp.float32)
    s128 = jnp.dot(s8, bw, preferred_element_type=jnp.float32)
    out = (msg + w128 * h) / (s128 + 1e-16) + bconv_ref[...]
    mu = jnp.mean(out, axis=1, keepdims=True)
    var = jnp.mean((out - mu) ** 2, axis=1, keepdims=True)
    nrm = (out - mu) / jnp.sqrt(var + 1e-5) * g_ref[...] + be_ref[...]
    o_ref[...] = (jnp.dot(nrm, wff_ref[...], preferred_element_type=jnp.float32)
                  + bff_ref[...] + x_ref[...])


def _stage3(pm, pw, h, as2, ad2, x, w_ff, b_ff, b_conv, g, be):
    return pl.pallas_call(
        _tc3_body,
        grid=(N // R,),
        in_specs=[
            pl.BlockSpec((NC, R, HC), lambda i: (0, i, 0)),
            pl.BlockSpec((NC, R, C), lambda i: (0, i, 0)),
            pl.BlockSpec((R, HC), lambda i: (i, 0)),
            pl.BlockSpec((R, C), lambda i: (i, 0)),
            pl.BlockSpec((R, C), lambda i: (i, 0)),
            pl.BlockSpec((R, F), lambda i: (i, 0)),
            pl.BlockSpec((HC, F), lambda i: (0, 0)),
            pl.BlockSpec((1, F), lambda i: (0, 0)),
            pl.BlockSpec((1, HC), lambda i: (0, 0)),
            pl.BlockSpec((1, HC), lambda i: (0, 0)),
            pl.BlockSpec((1, HC), lambda i: (0, 0)),
        ],
        out_specs=pl.BlockSpec((R, F), lambda i: (i, 0)),
        out_shape=jax.ShapeDtypeStruct((N, F), jnp.float32),
    )(pm, pw, h, as2, ad2, x, w_ff, b_ff, b_conv, g, be)


def kernel(x, edge_index, W_lin, att_src, att_dst, b_conv, ln_gamma,
           ln_beta, W_ff, b_ff):
    h, as2, ad2 = _stage1(x, W_lin,
                          att_src.reshape(1, HC), att_dst.reshape(1, HC))
    zm = jnp.zeros((RPT, HC), jnp.float32)
    zw = jnp.zeros((RPT, C), jnp.float32)
    ei4 = edge_index.reshape(2, NW, NCHUNK, B)
    pm, pw = _stage2(h, as2, ad2, ei4, zm, zw)
    return _stage3(pm, pw, h, as2, ad2, x, W_ff,
                   b_ff.reshape(1, F), b_conv.reshape(1, HC),
                   ln_gamma.reshape(1, HC), ln_beta.reshape(1, HC))


# speedup vs baseline: 1.1469x; 1.1469x over previous
import jax
import jax.numpy as jnp
from jax import lax
from jax.experimental import pallas as pl
from jax.experimental.pallas import tpu as pltpu
from jax.experimental.pallas import tpu_sc as plsc

N = 10000
E = 320000
F = 128
H = 8
C = 16
HC = H * C
R = 1000
NC = 2
NS = 16
NW = NC * NS
EPT = E // NW
B = 40
NCHUNK = EPT // B
NPAD = 10240
RPT = NPAD // NS


def _tc1_body(x_ref, w_ref, asrc_ref, adst_ref, h_ref, asrc2_ref, adst2_ref):
    h = jnp.dot(x_ref[...], w_ref[...], preferred_element_type=jnp.float32)
    rows = lax.broadcasted_iota(jnp.int32, (HC, H), 0) // C
    cols = lax.broadcasted_iota(jnp.int32, (HC, H), 1)
    b8 = (rows == cols).astype(jnp.float32)
    a_s = jnp.dot(h * asrc_ref[...], b8, preferred_element_type=jnp.float32)
    a_d = jnp.dot(h * adst_ref[...], b8, preferred_element_type=jnp.float32)
    h_ref[...] = h
    asrc2_ref[...] = jnp.concatenate([a_s, a_s], axis=1)
    adst2_ref[...] = jnp.concatenate([a_d, a_d], axis=1)


def _stage1(x, w_lin, att_src_flat, att_dst_flat):
    return pl.pallas_call(
        _tc1_body,
        grid=(N // R,),
        in_specs=[
            pl.BlockSpec((R, F), lambda i: (i, 0)),
            pl.BlockSpec((F, HC), lambda i: (0, 0)),
            pl.BlockSpec((1, HC), lambda i: (0, 0)),
            pl.BlockSpec((1, HC), lambda i: (0, 0)),
        ],
        out_specs=[
            pl.BlockSpec((R, HC), lambda i: (i, 0)),
            pl.BlockSpec((R, C), lambda i: (i, 0)),
            pl.BlockSpec((R, C), lambda i: (i, 0)),
        ],
        out_shape=[
            jax.ShapeDtypeStruct((N, HC), jnp.float32),
            jax.ShapeDtypeStruct((N, C), jnp.float32),
            jax.ShapeDtypeStruct((N, C), jnp.float32),
        ],
    )(x, w_lin, att_src_flat, att_dst_flat)


def _bcast_lane(v, j):
    idx = jnp.full((C,), j, jnp.int32)
    return v.at[idx].get(mode="promise_in_bounds")


def _sc_body(h_hbm, as2_hbm, ad2_hbm, ei_hbm, zm_hbm, zw_hbm, outm_hbm, outw_hbm,
             idxr, hring, asr, adr, wring, acc_m, acc_w,
             sg0, sg1, sg2, sg3, ss0, ss1, ss2, ss3,
             si0, si1, si2, si3, si4, si5, si6, si7):
    sem_g = [sg0, sg1, sg2, sg3]
    sem_s = [ss0, ss1, ss2, ss3]
    sem_i = [si0, si1, si2, si3, si4, si5, si6, si7]
    cid = lax.axis_index("c")
    sid = lax.axis_index("s")
    wid = sid * NC + cid

    pltpu.sync_copy(zm_hbm, acc_m.at[pl.ds(sid * RPT, RPT)])
    pltpu.sync_copy(zw_hbm, acc_w.at[pl.ds(sid * RPT, RPT)])
    plsc.subcore_barrier()

    def issue_i(c, j):
        pltpu.async_copy(ei_hbm.at[0, wid, c], idxr.at[j, 0], sem_i[j])
        pltpu.async_copy(ei_hbm.at[1, wid, c], idxr.at[j, 1], sem_i[j])

    def wait_i(c, j):
        pltpu.make_async_copy(ei_hbm.at[0, wid, c], idxr.at[j, 0],
                              sem_i[j]).wait()
        pltpu.make_async_copy(ei_hbm.at[1, wid, c], idxr.at[j, 1],
                              sem_i[j]).wait()

    def issue_g(j, b):
        pltpu.async_copy(h_hbm.at[idxr.at[j, 0]], hring.at[b], sem_g[b])
        pltpu.async_copy(as2_hbm.at[idxr.at[j, 0]], asr.at[b], sem_g[b])
        pltpu.async_copy(ad2_hbm.at[idxr.at[j, 1]], adr.at[b], sem_g[b])

    def wait_g(j, b):
        pltpu.make_async_copy(h_hbm.at[idxr.at[j, 0]], hring.at[b],
                              sem_g[b]).wait()
        pltpu.make_async_copy(as2_hbm.at[idxr.at[j, 0]], asr.at[b],
                              sem_g[b]).wait()
        pltpu.make_async_copy(ad2_hbm.at[idxr.at[j, 1]], adr.at[b],
                              sem_g[b]).wait()

    def issue_s(j, b):
        pltpu.async_copy(hring.at[b], acc_m.at[idxr.at[j, 1]], sem_s[b],
                         add=True)
        pltpu.async_copy(wring.at[b], acc_w.at[idxr.at[j, 1]], sem_s[b],
                         add=True)

    def wait_s(j, b):
        pltpu.make_async_copy(hring.at[b], acc_m.at[idxr.at[j, 1]],
                              sem_s[b]).wait()
        pltpu.make_async_copy(wring.at[b], acc_w.at[idxr.at[j, 1]],
                              sem_s[b]).wait()

    def compute(b):
        @plsc.parallel_loop(0, B, unroll=4)
        def edge_body(e):
            logit = asr[b, e, :] + adr[b, e, :]
            logit = jnp.where(logit > 0, logit, 0.2 * logit)
            w = jnp.exp(logit)
            wring[b, e, :] = w
            for hd in range(H):
                bc = _bcast_lane(w, hd)
                hring[b, e, pl.ds(hd * C, C)] = hring[b, e, pl.ds(hd * C, C)] * bc

    def step(c, m, first=False, do_g=True, do_i=True):
        b = m % 4
        wait_g(m % 8, b)
        compute(b)
        issue_s(m % 8, b)
        if not first:
            wait_s((m - 1) % 8, (m - 1) % 4)
        if do_g:
            wait_i(c + 3, (m + 3) % 8)
            issue_g((m + 3) % 8, (m + 3) % 4)
        if do_i:
            issue_i(c + 6, (m + 6) % 8)

    for c in range(6):
        issue_i(c, c)
    for c in range(3):
        wait_i(c, c)
        issue_g(c, c)
    step(0, 0, first=True)
    for c in range(1, 8):
        step(c, c)

    def octo(i, carry):
        for o in range(8):
            step(8 * i + o, o)
        return carry

    lax.fori_loop(1, (NCHUNK - 10) // 8, octo, 0)

    for c in range(NCHUNK - 10, NCHUNK):
        step(c, c % 8, do_g=(c + 3 <= NCHUNK - 1), do_i=(c + 6 <= NCHUNK - 1))
    wait_s((NCHUNK - 1) % 8, (NCHUNK - 1) % 4)
    plsc.subcore_barrier()

    for k in range(RPT // B):
        r0 = sid * RPT + k * B
        pltpu.sync_copy(acc_m.at[pl.ds(r0, B)], hring.at[k % 4])
        pltpu.sync_copy(hring.at[k % 4], outm_hbm.at[cid, pl.ds(r0, B)])
        pltpu.sync_copy(acc_w.at[pl.ds(r0, B)], wring.at[k % 4])
        pltpu.sync_copy(wring.at[k % 4], outw_hbm.at[cid, pl.ds(r0, B)])


def _stage2(h, as2, ad2, ei, zm, zw):
    mesh = plsc.VectorSubcoreMesh(core_axis_name="c", subcore_axis_name="s")
    k = pl.kernel(
        _sc_body,
        out_type=[
            jax.ShapeDtypeStruct((NC, NPAD, HC), jnp.float32),
            jax.ShapeDtypeStruct((NC, NPAD, C), jnp.float32),
        ],
        mesh=mesh,
        compiler_params=pltpu.CompilerParams(use_tc_tiling_on_sc=False),
        scratch_types=[
            pltpu.VMEM((8, 2, B), jnp.int32),
            pltpu.VMEM((4, B, HC), jnp.float32),
            pltpu.VMEM((4, B, C), jnp.float32),
            pltpu.VMEM((4, B, C), jnp.float32),
            pltpu.VMEM((4, B, C), jnp.float32),
            pltpu.VMEM_SHARED((NPAD, HC), jnp.float32),
            pltpu.VMEM_SHARED((NPAD, C), jnp.float32),
        ] + [pltpu.SemaphoreType.DMA] * 16,
    )
    return k(h, as2, ad2, ei, zm, zw)


def _tc3_body(pm_ref, pw_ref, h_ref, as2_ref, ad2_ref, x_ref, wff_ref, bff_ref,
              bconv_ref, g_ref, be_ref, o_ref):
    msg = pm_ref[0] + pm_ref[1]
    h = h_ref[...]
    l8 = as2_ref[:, :H] + ad2_ref[:, :H]
    w8 = jnp.exp(jnp.where(l8 > 0, l8, 0.2 * l8))
    s8 = pw_ref[0, :, :H] + pw_ref[1, :, :H] + w8
    rows_i = lax.broadcasted_iota(jnp.int32, (H, HC), 0)
    cols_i = lax.broadcasted_iota(jnp.int32, (H, HC), 1) // C
    bw = (rows_i == cols_i).astype(jnp.float32)
    w128 = jnp.dot(w8, bw, preferred_element_type=jnp.float32)
    s128 = jnp.dot(s8, bw, preferred_element_type=jnp.float32)
    out = (msg + w128 * h) / (s128 + 1e-16) + bconv_ref[...]
    mu = jnp.mean(out, axis=1, keepdims=True)
    var = jnp.mean((out - mu) ** 2, axis=1, keepdims=True)
    nrm = (out - mu) / jnp.sqrt(var + 1e-5) * g_ref[...] + be_ref[...]
    o_ref[...] = (jnp.dot(nrm, wff_ref[...], preferred_element_type=jnp.float32)
                  + bff_ref[...] + x_ref[...])


def _stage3(pm, pw, h, as2, ad2, x, w_ff, b_ff, b_conv, g, be):
    return pl.pallas_call(
        _tc3_body,
        grid=(N // R,),
        in_specs=[
            pl.BlockSpec((NC, R, HC), lambda i: (0, i, 0)),
            pl.BlockSpec((NC, R, C), lambda i: (0, i, 0)),
            pl.BlockSpec((R, HC), lambda i: (i, 0)),
            pl.BlockSpec((R, C), lambda i: (i, 0)),
            pl.BlockSpec((R, C), lambda i: (i, 0)),
            pl.BlockSpec((R, F), lambda i: (i, 0)),
            pl.BlockSpec((HC, F), lambda i: (0, 0)),
            pl.BlockSpec((1, F), lambda i: (0, 0)),
            pl.BlockSpec((1, HC), lambda i: (0, 0)),
            pl.BlockSpec((1, HC), lambda i: (0, 0)),
            pl.BlockSpec((1, HC), lambda i: (0, 0)),
        ],
        out_specs=pl.BlockSpec((R, F), lambda i: (i, 0)),
        out_shape=jax.ShapeDtypeStruct((N, F), jnp.float32),
    )(pm, pw, h, as2, ad2, x, w_ff, b_ff, b_conv, g, be)


def kernel(x, edge_index, W_lin, att_src, att_dst, b_conv, ln_gamma,
           ln_beta, W_ff, b_ff):
    h, as2, ad2 = _stage1(x, W_lin,
                          att_src.reshape(1, HC), att_dst.reshape(1, HC))
    zm = jnp.zeros((RPT, HC), jnp.float32)
    zw = jnp.zeros((RPT, C), jnp.float32)
    ei4 = edge_index.reshape(2, NW, NCHUNK, B)
    pm, pw = _stage2(h, as2, ad2, ei4, zm, zw)
    return _stage3(pm, pw, h, as2, ad2, x, W_ff,
                   b_ff.reshape(1, F), b_conv.reshape(1, HC),
                   ln_gamma.reshape(1, HC), ln_beta.reshape(1, HC))
